# baseline (speedup 1.0000x reference)
import functools

import jax
import jax.numpy as jnp
import numpy as np
from jax import lax
from jax.experimental import pallas as pl
from jax.experimental.pallas import tpu as pltpu
from jax.experimental.pallas import tpu_sc as plsc

N = 10000
E = 320000
D = 128
NP = 10240
EP = 327680
NROW = NP // 128
EROW = EP // 128
NWORK = 32
RPW = EROW // NWORK
NPT = NP // 16

F32 = jnp.float32
I32 = jnp.int32


def _mesh():
    return plsc.VectorSubcoreMesh(core_axis_name="c", subcore_axis_name="s")


def _seg1_body(table_h, src_h, dst_h, out_h, table_v, src_v, dst_v, vals_v,
               z_v, acc_sh):
    c = lax.axis_index("c")
    s = lax.axis_index("s")
    base = (c * 16 + s) * RPW
    pltpu.sync_copy(table_h, table_v)
    pltpu.sync_copy(src_h.at[pl.ds(base, RPW)], src_v)
    pltpu.sync_copy(dst_h.at[pl.ds(base, RPW)], dst_v)
    for i in range(NPT // 16):
        z_v[pl.ds(i * 16, 16)] = jnp.zeros((16,), F32)
    pltpu.sync_copy(z_v, acc_sh.at[pl.ds(s * NPT, NPT)])
    plsc.subcore_barrier()

    def body(g, carry):
        for j in range(8):
            idx16 = src_v[g, pl.ds(j * 16, 16)]
            vals_v[pl.ds(j * 16, 16)] = plsc.load_gather(
                table_v, [lax.shift_right_logical(idx16, 7),
                          lax.bitwise_and(idx16, 127)])
        pltpu.sync_copy(vals_v, acc_sh.at[dst_v.at[g]], add=True)
        return carry

    lax.fori_loop(0, RPW, body, 0)
    plsc.subcore_barrier()
    pltpu.sync_copy(acc_sh.at[pl.ds(s * NPT, NPT)],
                    out_h.at[c, pl.ds(s * NPT, NPT)])


def _seg1(table, src2d, dst2d):
    k = pl.kernel(
        _seg1_body,
        out_type=jax.ShapeDtypeStruct((2, NP), F32),
        mesh=_mesh(),
        compiler_params=pltpu.CompilerParams(needs_layout_passes=False),
        scratch_types=[
            pltpu.VMEM((NROW, 128), F32),
            pltpu.VMEM((RPW, 128), I32),
            pltpu.VMEM((RPW, 128), I32),
            pltpu.VMEM((128,), F32),
            pltpu.VMEM((NPT,), F32),
            pltpu.VMEM_SHARED((NP,), F32),
        ],
    )
    return k(table, src2d, dst2d)


DH = D // 2


def _seg128_body(y_h, zr_h, src_h, dst_h, out_h,
                 src_v, dst_v, rows_v, acc_sh):
    c = lax.axis_index("c")
    s = lax.axis_index("s")
    base = (c * 16 + s) * RPW
    pltpu.sync_copy(src_h.at[pl.ds(base, RPW)], src_v)
    pltpu.sync_copy(dst_h.at[pl.ds(base, RPW)], dst_v)
    pltpu.sync_copy(zr_h, acc_sh.at[pl.ds(s * NPT, NPT)])
    plsc.subcore_barrier()

    def body(g, carry):
        pltpu.sync_copy(y_h.at[src_v.at[g]], rows_v)
        pltpu.sync_copy(rows_v, acc_sh.at[dst_v.at[g]], add=True)
        return carry

    lax.fori_loop(0, RPW, body, 0)
    plsc.subcore_barrier()
    pltpu.sync_copy(acc_sh.at[pl.ds(s * NPT, NPT)],
                    out_h.at[c, pl.ds(s * NPT, NPT)])


def _seg128(y, src2d, dst2d, zrows):
    k = pl.kernel(
        _seg128_body,
        out_type=jax.ShapeDtypeStruct((2, NP, D), F32),
        mesh=_mesh(),
        compiler_params=pltpu.CompilerParams(needs_layout_passes=False),
        scratch_types=[
            pltpu.VMEM((RPW, 128), I32),
            pltpu.VMEM((RPW, 128), I32),
            pltpu.VMEM((128, D), F32),
            pltpu.VMEM_SHARED((NP, D), F32),
        ],
    )
    return k(y, zrows, src2d, dst2d)


def _elu(x):
    xn = jnp.minimum(x, 0.0)
    em1 = jnp.tanh(xn * 0.5) * (jnp.exp(xn) + 1.0)
    return jnp.where(x > 0, x, em1)


def _tca_mm_body(x_ref, am_ref, w_ref, g_ref, m_ref, xw_ref, y_ref, dis_ref):
    deg = m_ref[...] * (g_ref[0] + g_ref[1] + 2.0)
    dis = jnp.where(deg > 0, 1.0 / jnp.sqrt(jnp.maximum(deg, 1e-12)), 0.0)
    dis_ref[...] = dis
    xw = jnp.dot(x_ref[...] * am_ref[...], w_ref[...],
                 preferred_element_type=F32)
    xw_ref[...] = xw
    y_ref[...] = xw * dis


def _tca_mm(x, amc, w, gp, maskc):
    fo = w.shape[1]
    return pl.pallas_call(
        _tca_mm_body,
        out_shape=(jax.ShapeDtypeStruct((NP, fo), F32),
                   jax.ShapeDtypeStruct((NP, fo), F32),
                   jax.ShapeDtypeStruct((NP, 1), F32)),
    )(x, amc, w, gp.reshape(2, NP, 1), maskc)


def _tca_post_body(e_ref, dis_ref, xw_ref, m_ref, b_ref, wp_ref,
                   h_ref, s_ref, ys_ref):
    dis = dis_ref[...]
    m = m_ref[...]
    agg = dis * (e_ref[0] + e_ref[1]) + xw_ref[...] * ((dis * (2.0 * m)) * dis)
    h = _elu((agg + b_ref[...]) * m)
    h_ref[...] = h
    s = jnp.dot(h, wp_ref[...], preferred_element_type=F32)
    s_ref[...] = s
    ys_ref[...] = s * dis


def _tca_post(ep, disc, xw, maskc, b, wp):
    return pl.pallas_call(
        _tca_post_body,
        out_shape=(jax.ShapeDtypeStruct((NP, D), F32),
                   jax.ShapeDtypeStruct((NP, 1), F32),
                   jax.ShapeDtypeStruct((NP, 1), F32)),
        compiler_params=pltpu.CompilerParams(
            vmem_limit_bytes=100 * 1024 * 1024),
    )(ep, disc, xw, maskc, b.reshape(1, D), wp)


def _sortable_u32(x):
    ub = lax.bitcast_convert_type(x, jnp.uint32)
    return jnp.where(ub >= np.uint32(0x80000000), ~ub,
                     ub | np.uint32(0x80000000))


def _topk_mask(masked, k):
    u = _sortable_u32(masked)
    idx = (lax.broadcasted_iota(I32, (NROW, 128), 0) * 128
           + lax.broadcasted_iota(I32, (NROW, 128), 1))
    t = np.uint32(0)
    for b in range(31, -1, -1):
        cand = t | (np.uint32(1) << np.uint32(b))
        cnt = jnp.sum((u >= cand).astype(I32))
        t = jnp.where(cnt >= k, cand, t)
    cnt_gt = jnp.sum((u > t).astype(I32))
    r = k - cnt_gt
    eq = u == t
    xcut = jnp.int32(0)
    for b in range(13, -1, -1):
        cand = xcut + (1 << b)
        cnt = jnp.sum((eq & (idx < cand)).astype(I32))
        xcut = jnp.where(cnt < r, cand, xcut)
    return ((u > t) | (eq & (idx <= xcut))).astype(F32)


def _tcv_pool_body(k, es_ref, s_ref, dis_ref, m_ref, bp_ref,
                   mask_ref, am_ref):
    dis = dis_ref[...]
    m = m_ref[...]
    score = (dis * (es_ref[0] + es_ref[1])
             + s_ref[...] * ((dis * (2.0 * m)) * dis) + bp_ref[0, 0]) * m
    attn = jnp.tanh(score)
    masked = jnp.where(m > 0, attn, -jnp.inf)
    mk = _topk_mask(masked, k)
    mask_ref[...] = mk
    am_ref[...] = attn * mk


def _tcv_pool(esp, s80, dis80, mask80, bp, k):
    return pl.pallas_call(
        functools.partial(_tcv_pool_body, k),
        out_shape=(jax.ShapeDtypeStruct((NROW, 128), F32),
                   jax.ShapeDtypeStruct((NROW, 128), F32)),
    )(esp.reshape(2, NROW, 128), s80, dis80, mask80, bp.reshape(1, 1))


def _tcv_final_body(es_ref, dis_ref, s_ref, m_ref, b3_ref, c1w_ref, c1b_ref,
                    c2w_ref, out_ref):
    dis = dis_ref[...]
    m = m_ref[...]
    h3 = (dis * (es_ref[0] + es_ref[1])
          + s_ref[...] * ((dis * (2.0 * m)) * dis) + b3_ref[0, 0]) * m
    key = jnp.where(m > 0, h3, -jnp.inf)
    idx = (lax.broadcasted_iota(I32, (NROW, 128), 0) * 128
           + lax.broadcasted_iota(I32, (NROW, 128), 1))
    pooled = jnp.zeros((1, 32), F32)
    sel32 = lax.broadcasted_iota(I32, (1, 32), 1)
    for j in range(30):
        mx = jnp.max(key)
        first = jnp.min(jnp.where(key == mx, idx, jnp.int32(NP)))
        pooled = pooled + mx * (sel32 == j).astype(F32)
        key = jnp.where(idx == first, -jnp.inf, key)
    hid = _elu(jnp.dot(pooled, c1w_ref[...], preferred_element_type=F32)
               + c1b_ref[...])
    out_ref[...] = jnp.dot(hid, c2w_ref[...], preferred_element_type=F32)


def _tcv_final(esp, dis80, s80, mask80, b3, c1wp, c1b, c2wp):
    return pl.pallas_call(
        _tcv_final_body,
        out_shape=jax.ShapeDtypeStruct((1, 128), F32),
    )(esp.reshape(2, NROW, 128), dis80, s80, mask80, b3.reshape(1, 1),
      c1wp, c1b.reshape(1, D), c2wp)


def kernel(x, edge_index, edge_attr, W1, b1, Wp1, bp1, W2, b2, Wp2, bp2,
           W3, b3, C1_W, C1_b, C2_W, C2_b):
    xp = jnp.pad(x, ((0, NP - N), (0, 0)))
    pad_idx = (N + (jnp.arange(EP - E, dtype=I32) % (NP - N))).astype(I32)
    src = jnp.concatenate([edge_index[0], pad_idx]).reshape(EROW, 128)
    dst = jnp.concatenate([edge_index[1], pad_idx]).reshape(EROW, 128)
    mask0 = (jnp.arange(NP) < N).astype(F32)
    mask0_80 = mask0.reshape(NROW, 128)
    zrows = jnp.zeros((NPT, D), F32)
    c1wp = jnp.pad(C1_W, ((0, 2), (0, 0)))
    c2wp = jnp.pad(C2_W, ((0, 0), (0, 118)))

    g1p = _seg1(mask0_80, src, dst)
    xw1, y1, dis1c = _tca_mm(xp, mask0.reshape(NP, 1), W1, g1p,
                             mask0.reshape(NP, 1))
    dis1_80 = dis1c.reshape(NROW, 128)
    e1p = _seg128(y1, src, dst, zrows)
    h, s1c, ys1c = _tca_post(e1p, dis1c, xw1, mask0.reshape(NP, 1), b1, Wp1)
    es1p = _seg1(ys1c.reshape(NROW, 128), src, dst)
    mask1_80, am1_80 = _tcv_pool(es1p, s1c.reshape(NROW, 128), dis1_80,
                                 mask0_80, bp1, 5000)
    g2p = _seg1(mask1_80, src, dst)
    xw2, y2, dis2c = _tca_mm(h, am1_80.reshape(NP, 1), W2, g2p,
                             mask1_80.reshape(NP, 1))
    dis2_80 = dis2c.reshape(NROW, 128)
    e2p = _seg128(y2, src, dst, zrows)
    h2, s2c, ys2c = _tca_post(e2p, dis2c, xw2, mask1_80.reshape(NP, 1),
                              b2, Wp2)
    es2p = _seg1(ys2c.reshape(NROW, 128), src, dst)
    mask2_80, am2_80 = _tcv_pool(es2p, s2c.reshape(NROW, 128), dis2_80,
                                 mask1_80, bp2, 2500)
    g3p = _seg1(mask2_80, src, dst)
    s3c, ys3c, dis3c = _tca_mm(h2, am2_80.reshape(NP, 1), W3, g3p,
                               mask2_80.reshape(NP, 1))
    dis3_80 = dis3c.reshape(NROW, 128)
    e3p = _seg1(ys3c.reshape(NROW, 128), src, dst)
    out = _tcv_final(e3p, dis3_80, s3c.reshape(NROW, 128), mask2_80,
                     b3, c1wp, C1_b, c2wp)
    return out[:, :10]


# speedup vs baseline: 1.0385x; 1.0385x over previous
import functools

import jax
import jax.numpy as jnp
import numpy as np
from jax import lax
from jax.experimental import pallas as pl
from jax.experimental.pallas import tpu as pltpu
from jax.experimental.pallas import tpu_sc as plsc

N = 10000
E = 320000
D = 128
NP = 10240
EP = 327680
NROW = NP // 128
EROW = EP // 128
NWORK = 32
RPW = EROW // NWORK
NPT = NP // 16

F32 = jnp.float32
I32 = jnp.int32


def _mesh():
    return plsc.VectorSubcoreMesh(core_axis_name="c", subcore_axis_name="s")


def _seg1_body(table_h, src_h, dst_h, out_h, table_v, src_v, dst_v, vals_v,
               z_v, acc_sh):
    c = lax.axis_index("c")
    s = lax.axis_index("s")
    base = (c * 16 + s) * RPW
    pltpu.sync_copy(table_h, table_v)
    pltpu.sync_copy(src_h.at[pl.ds(base, RPW)], src_v)
    pltpu.sync_copy(dst_h.at[pl.ds(base, RPW)], dst_v)
    for i in range(NPT // 16):
        z_v[pl.ds(i * 16, 16)] = jnp.zeros((16,), F32)
    pltpu.sync_copy(z_v, acc_sh.at[pl.ds(s * NPT, NPT)])
    plsc.subcore_barrier()

    def body(g, carry):
        for j in range(8):
            idx16 = src_v[g, pl.ds(j * 16, 16)]
            vals_v[pl.ds(j * 16, 16)] = plsc.load_gather(
                table_v, [lax.shift_right_logical(idx16, 7),
                          lax.bitwise_and(idx16, 127)])
        pltpu.sync_copy(vals_v, acc_sh.at[dst_v.at[g]], add=True)
        return carry

    lax.fori_loop(0, RPW, body, 0)
    plsc.subcore_barrier()
    pltpu.sync_copy(acc_sh.at[pl.ds(s * NPT, NPT)],
                    out_h.at[c, pl.ds(s * NPT, NPT)])


def _seg1(table, src2d, dst2d):
    k = pl.kernel(
        _seg1_body,
        out_type=jax.ShapeDtypeStruct((2, NP), F32),
        mesh=_mesh(),
        compiler_params=pltpu.CompilerParams(needs_layout_passes=False),
        scratch_types=[
            pltpu.VMEM((NROW, 128), F32),
            pltpu.VMEM((RPW, 128), I32),
            pltpu.VMEM((RPW, 128), I32),
            pltpu.VMEM((128,), F32),
            pltpu.VMEM((NPT,), F32),
            pltpu.VMEM_SHARED((NP,), F32),
        ],
    )
    return k(table, src2d, dst2d)


DH = D // 2


def _seg128_body(y_h, zr_h, src_h, dst_h, out_h,
                 src_v, dst_v, rows_v, acc_sh):
    c = lax.axis_index("c")
    s = lax.axis_index("s")
    base = (c * 16 + s) * RPW
    pltpu.sync_copy(src_h.at[pl.ds(base, RPW)], src_v)
    pltpu.sync_copy(dst_h.at[pl.ds(base, RPW)], dst_v)
    pltpu.sync_copy(zr_h, acc_sh.at[pl.ds(s * NPT, NPT)])
    plsc.subcore_barrier()

    def body(g, carry):
        pltpu.sync_copy(y_h.at[src_v.at[g]], rows_v)
        pltpu.sync_copy(rows_v, acc_sh.at[dst_v.at[g]], add=True)
        return carry

    lax.fori_loop(0, RPW, body, 0)
    plsc.subcore_barrier()
    pltpu.sync_copy(acc_sh.at[pl.ds(s * NPT, NPT)],
                    out_h.at[c, pl.ds(s * NPT, NPT)])


def _seg128(y, src2d, dst2d, zrows):
    k = pl.kernel(
        _seg128_body,
        out_type=jax.ShapeDtypeStruct((2, NP, D), F32),
        mesh=_mesh(),
        compiler_params=pltpu.CompilerParams(needs_layout_passes=False),
        scratch_types=[
            pltpu.VMEM((RPW, 128), I32),
            pltpu.VMEM((RPW, 128), I32),
            pltpu.VMEM((128, D), F32),
            pltpu.VMEM_SHARED((NP, D), F32),
        ],
    )
    return k(y, zrows, src2d, dst2d)


def _elu(x):
    xn = jnp.minimum(x, 0.0)
    em1 = jnp.tanh(xn * 0.5) * (jnp.exp(xn) + 1.0)
    return jnp.where(x > 0, x, em1)


def _tcv_dis_body(g_ref, m_ref, dis_ref):
    deg = m_ref[...] * (g_ref[0] + g_ref[1] + 2.0)
    dis_ref[...] = jnp.where(deg > 0, 1.0 / jnp.sqrt(jnp.maximum(deg, 1e-12)),
                             0.0)


def _tcv_dis(gp, mask80):
    return pl.pallas_call(
        _tcv_dis_body,
        out_shape=jax.ShapeDtypeStruct((NROW, 128), F32),
    )(gp.reshape(2, NROW, 128), mask80)


def _tca_mm_body(x_ref, am_ref, w_ref, dis_ref, xw_ref, y_ref):
    xw = jnp.dot(x_ref[...] * am_ref[...], w_ref[...],
                 preferred_element_type=F32)
    xw_ref[...] = xw
    y_ref[...] = xw * dis_ref[...]


def _tca_mm(x, amc, w, disc):
    fo = w.shape[1]
    return pl.pallas_call(
        _tca_mm_body,
        out_shape=(jax.ShapeDtypeStruct((NP, fo), F32),
                   jax.ShapeDtypeStruct((NP, fo), F32)),
    )(x, amc, w, disc)


def _tca_post_body(e_ref, dis_ref, xw_ref, m_ref, b_ref, wp_ref,
                   h_ref, s_ref, ys_ref):
    dis = dis_ref[...]
    m = m_ref[...]
    agg = dis * (e_ref[0] + e_ref[1]) + xw_ref[...] * ((dis * (2.0 * m)) * dis)
    h = _elu((agg + b_ref[...]) * m)
    h_ref[...] = h
    s = jnp.dot(h, wp_ref[...], preferred_element_type=F32)
    s_ref[...] = s
    ys_ref[...] = s * dis


def _tca_post(ep, disc, xw, maskc, b, wp):
    return pl.pallas_call(
        _tca_post_body,
        out_shape=(jax.ShapeDtypeStruct((NP, D), F32),
                   jax.ShapeDtypeStruct((NP, 1), F32),
                   jax.ShapeDtypeStruct((NP, 1), F32)),
        compiler_params=pltpu.CompilerParams(
            vmem_limit_bytes=100 * 1024 * 1024),
    )(ep, disc, xw, maskc, b.reshape(1, D), wp)


def _sortable_u32(x):
    ub = lax.bitcast_convert_type(x, jnp.uint32)
    return jnp.where(ub >= np.uint32(0x80000000), ~ub,
                     ub | np.uint32(0x80000000))


def _topk_mask(masked, k):
    u = _sortable_u32(masked)
    idx = (lax.broadcasted_iota(I32, (NROW, 128), 0) * 128
           + lax.broadcasted_iota(I32, (NROW, 128), 1))
    t = np.uint32(0)
    for b in range(31, -1, -1):
        cand = t | (np.uint32(1) << np.uint32(b))
        cnt = jnp.sum((u >= cand).astype(I32))
        t = jnp.where(cnt >= k, cand, t)
    cnt_gt = jnp.sum((u > t).astype(I32))
    r = k - cnt_gt
    eq = u == t
    xcut = jnp.int32(0)
    for b in range(13, -1, -1):
        cand = xcut + (1 << b)
        cnt = jnp.sum((eq & (idx < cand)).astype(I32))
        xcut = jnp.where(cnt < r, cand, xcut)
    return ((u > t) | (eq & (idx <= xcut))).astype(F32)


def _tcv_pool_body(k, es_ref, s_ref, dis_ref, m_ref, bp_ref,
                   mask_ref, am_ref):
    dis = dis_ref[...]
    m = m_ref[...]
    score = (dis * (es_ref[0] + es_ref[1])
             + s_ref[...] * ((dis * (2.0 * m)) * dis) + bp_ref[0, 0]) * m
    attn = jnp.tanh(score)
    masked = jnp.where(m > 0, attn, -jnp.inf)
    mk = _topk_mask(masked, k)
    mask_ref[...] = mk
    am_ref[...] = attn * mk


def _tcv_pool(esp, s80, dis80, mask80, bp, k):
    return pl.pallas_call(
        functools.partial(_tcv_pool_body, k),
        out_shape=(jax.ShapeDtypeStruct((NROW, 128), F32),
                   jax.ShapeDtypeStruct((NROW, 128), F32)),
    )(esp.reshape(2, NROW, 128), s80, dis80, mask80, bp.reshape(1, 1))


def _tcv_final_body(es_ref, dis_ref, s_ref, m_ref, b3_ref, c1w_ref, c1b_ref,
                    c2w_ref, out_ref):
    dis = dis_ref[...]
    m = m_ref[...]
    h3 = (dis * (es_ref[0] + es_ref[1])
          + s_ref[...] * ((dis * (2.0 * m)) * dis) + b3_ref[0, 0]) * m
    key = jnp.where(m > 0, h3, -jnp.inf)
    idx = (lax.broadcasted_iota(I32, (NROW, 128), 0) * 128
           + lax.broadcasted_iota(I32, (NROW, 128), 1))
    pooled = jnp.zeros((1, 32), F32)
    sel32 = lax.broadcasted_iota(I32, (1, 32), 1)
    for j in range(30):
        mx = jnp.max(key)
        first = jnp.min(jnp.where(key == mx, idx, jnp.int32(NP)))
        pooled = pooled + mx * (sel32 == j).astype(F32)
        key = jnp.where(idx == first, -jnp.inf, key)
    hid = _elu(jnp.dot(pooled, c1w_ref[...], preferred_element_type=F32)
               + c1b_ref[...])
    out_ref[...] = jnp.dot(hid, c2w_ref[...], preferred_element_type=F32)


def _tcv_final(esp, dis80, s80, mask80, b3, c1wp, c1b, c2wp):
    return pl.pallas_call(
        _tcv_final_body,
        out_shape=jax.ShapeDtypeStruct((1, 128), F32),
    )(esp.reshape(2, NROW, 128), dis80, s80, mask80, b3.reshape(1, 1),
      c1wp, c1b.reshape(1, D), c2wp)


def kernel(x, edge_index, edge_attr, W1, b1, Wp1, bp1, W2, b2, Wp2, bp2,
           W3, b3, C1_W, C1_b, C2_W, C2_b):
    xp = jnp.pad(x, ((0, NP - N), (0, 0)))
    pad_idx = (N + (jnp.arange(EP - E, dtype=I32) % (NP - N))).astype(I32)
    src = jnp.concatenate([edge_index[0], pad_idx]).reshape(EROW, 128)
    dst = jnp.concatenate([edge_index[1], pad_idx]).reshape(EROW, 128)
    mask0 = (jnp.arange(NP) < N).astype(F32)
    mask0_80 = mask0.reshape(NROW, 128)
    zrows = jnp.zeros((NPT, D), F32)
    c1wp = jnp.pad(C1_W, ((0, 2), (0, 0)))
    c2wp = jnp.pad(C2_W, ((0, 0), (0, 118)))

    g1p = _seg1(mask0_80, src, dst)
    dis1_80 = _tcv_dis(g1p, mask0_80)
    dis1c = dis1_80.reshape(NP, 1)
    xw1, y1 = _tca_mm(xp, mask0.reshape(NP, 1), W1, dis1c)
    e1p = _seg128(y1, src, dst, zrows)
    h, s1c, ys1c = _tca_post(e1p, dis1c, xw1, mask0.reshape(NP, 1), b1, Wp1)
    es1p = _seg1(ys1c.reshape(NROW, 128), src, dst)
    mask1_80, am1_80 = _tcv_pool(es1p, s1c.reshape(NROW, 128), dis1_80,
                                 mask0_80, bp1, 5000)
    g2p = _seg1(mask1_80, src, dst)
    dis2_80 = _tcv_dis(g2p, mask1_80)
    dis2c = dis2_80.reshape(NP, 1)
    xw2, y2 = _tca_mm(h, am1_80.reshape(NP, 1), W2, dis2c)
    e2p = _seg128(y2, src, dst, zrows)
    h2, s2c, ys2c = _tca_post(e2p, dis2c, xw2, mask1_80.reshape(NP, 1),
                              b2, Wp2)
    es2p = _seg1(ys2c.reshape(NROW, 128), src, dst)
    mask2_80, am2_80 = _tcv_pool(es2p, s2c.reshape(NROW, 128), dis2_80,
                                 mask1_80, bp2, 2500)
    g3p = _seg1(mask2_80, src, dst)
    dis3_80 = _tcv_dis(g3p, mask2_80)
    dis3c = dis3_80.reshape(NP, 1)
    s3c, ys3c = _tca_mm(h2, am2_80.reshape(NP, 1), W3, dis3c)
    e3p = _seg1(ys3c.reshape(NROW, 128), src, dst)
    out = _tcv_final(e3p, dis3_80, s3c.reshape(NROW, 128), mask2_80,
                     b3, c1wp, C1_b, c2wp)
    return out[:, :10]
